# baseline (speedup 1.0000x reference)
import numpy as np
import jax
import jax.numpy as jnp
from jax.experimental import pallas as pl
from jax.experimental.pallas import tpu as pltpu

_DEEP = 256
_TYPE = 10
_PARAM = 12
_F = _DEEP + _TYPE + _PARAM

_api_lists = [[0], [0, 1], [1, 2], [3], [4, 5], [6], [7, 8], [9], [10], [11]]
_API_NP = np.zeros((_TYPE, _PARAM), dtype=np.float32)
for _t, _lst in enumerate(_api_lists):
    for _p in _lst:
        _API_NP[_t, _p] = 1.0

_ROWS = 2048
_BB = 1
_K = 4


def _partial_sums(x, t, api):
    pad = t[:, _DEEP:_DEEP + 1] == -1.0
    validf = jnp.where(pad, 0.0, 1.0)[:, 0]

    xs = x[:, :_DEEP]
    ts = t[:, :_DEEP]
    ds = jnp.where(pad, -100.0 - ts, xs - ts)
    s_svg = jnp.sum(ds * ds)

    xt = x[:, _DEEP:_DEEP + _TYPE]
    tt = t[:, _DEEP:_DEEP + _TYPE]
    m = jnp.max(xt, axis=1, keepdims=True)
    lse = m[:, 0] + jnp.log(jnp.sum(jnp.exp(xt - m), axis=1))
    picked = jnp.sum(xt * tt, axis=1)
    s_type = jnp.sum((lse - picked) * validf)
    cnt = jnp.sum(validf)

    xp = x[:, _DEEP + _TYPE:]
    tp = t[:, _DEEP + _TYPE:]
    copy = jax.lax.dot(tt, api, preferred_element_type=jnp.float32) > 0.5
    dp = jnp.where(pad, -100.0 - tp, jnp.where(copy, 0.0, xp - tp))
    s_param = jnp.sum(dp * dp)
    return s_svg, s_type, cnt, s_param


def _body(*refs):
    o_ref = refs[-1]
    api_ref = refs[-2]
    xrefs = refs[:_K]
    trefs = refs[_K:2 * _K]
    i = pl.program_id(0)
    j = pl.program_id(1)

    s_svg = 0.0
    s_type = 0.0
    cnt = 0.0
    s_param = 0.0
    for k in range(_K):
        x = xrefs[k][...].reshape(_BB * _ROWS, _F)
        t = trefs[k][...].reshape(_BB * _ROWS, _F)
        a, b, c, d = _partial_sums(x, t, api_ref[...])
        s_svg += a
        s_type += b
        cnt += c
        s_param += d

    @pl.when((i == 0) & (j == 0))
    def _init():
        o_ref[0] = 0.0
        o_ref[1] = 0.0
        o_ref[2] = 0.0
        o_ref[3] = 0.0

    o_ref[0] += s_svg
    o_ref[1] += s_type
    o_ref[2] += cnt
    o_ref[3] += s_param


def kernel(input, target, target_padding_mask):
    b, s, _ = input.shape
    n = b * s
    bi = b // _K
    gi = bi // _BB
    gj = s // _ROWS

    def _spec(k):
        return pl.BlockSpec((_BB, _ROWS, _F),
                            lambda i, j, k=k: (k * gi + i, j, 0))

    specs = ([_spec(k) for k in range(_K)] * 2
             + [pl.BlockSpec((_TYPE, _PARAM), lambda i, j: (0, 0))])

    sums = pl.pallas_call(
        _body,
        grid=(gi, gj),
        in_specs=specs,
        out_specs=pl.BlockSpec(memory_space=pltpu.SMEM),
        out_shape=jax.ShapeDtypeStruct((4,), jnp.float32),
    )(*([input] * _K + [target] * _K + [jnp.asarray(_API_NP)]))
    loss = (10.0 * sums[0] / (n * _DEEP)
            + 0.1 * sums[1] / jnp.maximum(sums[2], 1.0)
            + sums[3] / (n * _PARAM))
    return loss
